# Initial kernel scaffold; baseline (speedup 1.0000x reference)
import functools

import jax
import jax.numpy as jnp
from jax import lax
from jax.experimental import pallas as pl
from jax.experimental.pallas import tpu as pltpu
from jax.experimental.pallas import tpu_sc as plsc

N = 10000
E = 320000
D_IN = 128
D_HID = 256
DH = 128

CH = 128
NCHUNKS = E // CH
NC, NS = 2, 16
NTILES = NC * NS
RPS = N // NS

_MESH = plsc.VectorSubcoreMesh(
    core_axis_name="c", subcore_axis_name="s", num_cores=NC, num_subcores=NS)

_f32 = jnp.float32


@functools.partial(
    pl.kernel,
    out_type=(
        jax.ShapeDtypeStruct((N, D_IN), _f32),
        jax.ShapeDtypeStruct((N, D_IN), _f32),
        jax.ShapeDtypeStruct((N, 8), _f32),
        jax.ShapeDtypeStruct((N, 8), _f32),
    ),
    mesh=_MESH,
    scratch_types=[
        pltpu.VMEM((CH,), jnp.int32),
        pltpu.VMEM((CH,), jnp.int32),
        pltpu.VMEM((CH, D_IN), _f32),
        pltpu.VMEM((CH, 8), _f32),
        pltpu.VMEM_SHARED((N, D_IN), _f32),
        pltpu.VMEM_SHARED((N, 8), _f32),
        pltpu.SemaphoreType.DMA,
    ],
)
def _sc_agg_l1(x_h, src_h, dst_h, z128_h, z8_h, ones_h,
               agg0_h, agg1_h, cnt0_h, cnt1_h,
               sidx_v, didx_v, rows_v, ones_v, acc_s, cnt_s, sem):
    c = lax.axis_index("c")
    s = lax.axis_index("s")
    wid = c * NS + s
    base = s * RPS
    pltpu.sync_copy(z128_h, acc_s.at[pl.ds(base, RPS)])
    pltpu.sync_copy(z8_h, cnt_s.at[pl.ds(base, RPS)])
    pltpu.sync_copy(ones_h, ones_v)
    plsc.subcore_barrier()

    nmine = jnp.where(wid < NCHUNKS - (NCHUNKS // NTILES) * NTILES,
                      NCHUNKS // NTILES + 1, NCHUNKS // NTILES)

    def body(i, carry):
        j = wid + i * NTILES
        eb = j * CH
        pltpu.sync_copy(src_h.at[pl.ds(eb, CH)], sidx_v)
        pltpu.sync_copy(dst_h.at[pl.ds(eb, CH)], didx_v)
        pltpu.async_copy(x_h.at[sidx_v], rows_v, sem).wait()
        pltpu.sync_copy(rows_v, acc_s.at[didx_v], add=True)
        pltpu.sync_copy(ones_v, cnt_s.at[didx_v], add=True)
        return carry

    lax.fori_loop(0, nmine, body, 0)
    plsc.subcore_barrier()

    @pl.when(c == 0)
    def _():
        pltpu.sync_copy(acc_s.at[pl.ds(base, RPS)], agg0_h.at[pl.ds(base, RPS)])
        pltpu.sync_copy(cnt_s.at[pl.ds(base, RPS)], cnt0_h.at[pl.ds(base, RPS)])

    @pl.when(c == 1)
    def _():
        pltpu.sync_copy(acc_s.at[pl.ds(base, RPS)], agg1_h.at[pl.ds(base, RPS)])
        pltpu.sync_copy(cnt_s.at[pl.ds(base, RPS)], cnt1_h.at[pl.ds(base, RPS)])


@functools.partial(
    pl.kernel,
    out_type=(
        jax.ShapeDtypeStruct((N, DH), _f32),
        jax.ShapeDtypeStruct((N, DH), _f32),
    ),
    mesh=_MESH,
    scratch_types=[
        pltpu.VMEM((CH,), jnp.int32),
        pltpu.VMEM((CH,), jnp.int32),
        pltpu.VMEM((CH, DH), _f32),
        pltpu.VMEM_SHARED((N, DH), _f32),
        pltpu.SemaphoreType.DMA,
    ],
)
def _sc_agg_l2(h1a_h, h1b_h, src_h, dst_h, z128_h,
               agg2a_h, agg2b_h,
               sidx_v, didx_v, rows_v, acc_s, sem):
    c = lax.axis_index("c")
    s = lax.axis_index("s")
    base = s * RPS
    pltpu.sync_copy(z128_h, acc_s.at[pl.ds(base, RPS)])
    plsc.subcore_barrier()

    nmine = jnp.where(s < NCHUNKS - (NCHUNKS // NS) * NS,
                      NCHUNKS // NS + 1, NCHUNKS // NS)

    def run(table_h, out_h):
        def body(i, carry):
            j = s + i * NS
            eb = j * CH
            pltpu.sync_copy(src_h.at[pl.ds(eb, CH)], sidx_v)
            pltpu.sync_copy(dst_h.at[pl.ds(eb, CH)], didx_v)
            pltpu.async_copy(table_h.at[sidx_v], rows_v, sem).wait()
            pltpu.sync_copy(rows_v, acc_s.at[didx_v], add=True)
            return carry

        lax.fori_loop(0, nmine, body, 0)
        plsc.subcore_barrier()
        pltpu.sync_copy(acc_s.at[pl.ds(base, RPS)], out_h.at[pl.ds(base, RPS)])

    @pl.when(c == 0)
    def _():
        run(h1a_h, agg2a_h)

    @pl.when(c == 1)
    def _():
        run(h1b_h, agg2b_h)


@functools.partial(
    pl.kernel,
    out_type=(
        jax.ShapeDtypeStruct((NCHUNKS, CH), _f32),
        jax.ShapeDtypeStruct((NCHUNKS, CH), _f32),
    ),
    mesh=_MESH,
    scratch_types=[
        pltpu.VMEM((CH,), jnp.int32),
        pltpu.VMEM((CH,), jnp.int32),
        pltpu.VMEM((CH, DH), _f32),
        pltpu.VMEM((CH, DH), _f32),
        pltpu.VMEM((CH,), _f32),
        pltpu.SemaphoreType.DMA,
        pltpu.SemaphoreType.DMA,
    ],
)
def _sc_edge_dot(h2a_h, h2b_h, ga_h, gb_h, src_h, dst_h,
                 pa_h, pb_h,
                 sidx_v, didx_v, hrows_v, grows_v, res_v, sem0, sem1):
    c = lax.axis_index("c")
    s = lax.axis_index("s")

    nmine = jnp.where(s < NCHUNKS - (NCHUNKS // NS) * NS,
                      NCHUNKS // NS + 1, NCHUNKS // NS)

    def run(h_h, g_h, out_h):
        def body(i, carry):
            j = s + i * NS
            eb = j * CH
            pltpu.sync_copy(src_h.at[pl.ds(eb, CH)], sidx_v)
            pltpu.sync_copy(dst_h.at[pl.ds(eb, CH)], didx_v)
            cp0 = pltpu.async_copy(h_h.at[sidx_v], hrows_v, sem0)
            cp1 = pltpu.async_copy(g_h.at[didx_v], grows_v, sem1)
            cp0.wait()
            cp1.wait()

            def edge(e, carry2):
                acc = hrows_v[e, pl.ds(0, 16)] * grows_v[e, pl.ds(0, 16)]
                for q in range(1, DH // 16):
                    acc = acc + (hrows_v[e, pl.ds(q * 16, 16)] *
                                 grows_v[e, pl.ds(q * 16, 16)])
                res_v[e] = jnp.sum(acc)
                return carry2

            lax.fori_loop(0, CH, edge, 0)
            pltpu.sync_copy(res_v, out_h.at[j])
            return carry

        lax.fori_loop(0, nmine, body, 0)

    @pl.when(c == 0)
    def _():
        run(h2a_h, ga_h, pa_h)

    @pl.when(c == 1)
    def _():
        run(h2b_h, gb_h, pb_h)


_NB = 400


def _tc1_body(agg0_r, agg1_r, cnt0_r, cnt1_r, x_r, wl_r, b_r, wr_r,
              h1a_r, h1b_r):
    cnt = cnt0_r[:, 0:1] + cnt1_r[:, 0:1]
    agg = agg0_r[...] + agg1_r[...]
    mean = agg / jnp.maximum(cnt, 1.0)
    h = (jnp.dot(mean, wl_r[...], preferred_element_type=_f32,
                 precision="highest")
         + jnp.dot(x_r[...], wr_r[...], preferred_element_type=_f32,
                   precision="highest")
         + b_r[...])
    h1a_r[...] = h[:, :DH]
    h1b_r[...] = h[:, DH:]


def _tc_layer1(agg0, agg1, cnt0, cnt1, x, W1_l, b1, W1_r):
    grid = N // _NB
    return pl.pallas_call(
        _tc1_body,
        grid=(grid,),
        in_specs=[
            pl.BlockSpec((_NB, D_IN), lambda i: (i, 0)),
            pl.BlockSpec((_NB, D_IN), lambda i: (i, 0)),
            pl.BlockSpec((_NB, 8), lambda i: (i, 0)),
            pl.BlockSpec((_NB, 8), lambda i: (i, 0)),
            pl.BlockSpec((_NB, D_IN), lambda i: (i, 0)),
            pl.BlockSpec((D_IN, D_HID), lambda i: (0, 0)),
            pl.BlockSpec((1, D_HID), lambda i: (0, 0)),
            pl.BlockSpec((D_IN, D_HID), lambda i: (0, 0)),
        ],
        out_specs=[
            pl.BlockSpec((_NB, DH), lambda i: (i, 0)),
            pl.BlockSpec((_NB, DH), lambda i: (i, 0)),
        ],
        out_shape=[
            jax.ShapeDtypeStruct((N, DH), _f32),
            jax.ShapeDtypeStruct((N, DH), _f32),
        ],
    )(agg0, agg1, cnt0, cnt1, x, W1_l, b1, W1_r)


def _tc2_body(a2a_r, a2b_r, cnt0_r, cnt1_r, h1a_r, h1b_r, wl_r, b_r, wr_r,
              fcr_r, h2a_r, h2b_r, ga_r, gb_r):
    cnt = jnp.maximum(cnt0_r[:, 0:1] + cnt1_r[:, 0:1], 1.0)
    mlo = a2a_r[...] / cnt
    mhi = a2b_r[...] / cnt
    kw = dict(preferred_element_type=_f32, precision="highest")
    h = (jnp.dot(mlo, wl_r[0:DH, :], **kw)
         + jnp.dot(mhi, wl_r[DH:, :], **kw)
         + jnp.dot(h1a_r[...], wr_r[0:DH, :], **kw)
         + jnp.dot(h1b_r[...], wr_r[DH:, :], **kw)
         + b_r[...])
    g = h * fcr_r[...]
    h2a_r[...] = h[:, :DH]
    h2b_r[...] = h[:, DH:]
    ga_r[...] = g[:, :DH]
    gb_r[...] = g[:, DH:]


def _tc_layer2(a2a, a2b, cnt0, cnt1, h1a, h1b, W2_l, b2, W2_r, fc_row):
    grid = N // _NB
    return pl.pallas_call(
        _tc2_body,
        grid=(grid,),
        in_specs=[
            pl.BlockSpec((_NB, DH), lambda i: (i, 0)),
            pl.BlockSpec((_NB, DH), lambda i: (i, 0)),
            pl.BlockSpec((_NB, 8), lambda i: (i, 0)),
            pl.BlockSpec((_NB, 8), lambda i: (i, 0)),
            pl.BlockSpec((_NB, DH), lambda i: (i, 0)),
            pl.BlockSpec((_NB, DH), lambda i: (i, 0)),
            pl.BlockSpec((D_HID, D_HID), lambda i: (0, 0)),
            pl.BlockSpec((1, D_HID), lambda i: (0, 0)),
            pl.BlockSpec((D_HID, D_HID), lambda i: (0, 0)),
            pl.BlockSpec((1, D_HID), lambda i: (0, 0)),
        ],
        out_specs=[
            pl.BlockSpec((_NB, DH), lambda i: (i, 0)),
            pl.BlockSpec((_NB, DH), lambda i: (i, 0)),
            pl.BlockSpec((_NB, DH), lambda i: (i, 0)),
            pl.BlockSpec((_NB, DH), lambda i: (i, 0)),
        ],
        out_shape=[
            jax.ShapeDtypeStruct((N, DH), _f32),
            jax.ShapeDtypeStruct((N, DH), _f32),
            jax.ShapeDtypeStruct((N, DH), _f32),
            jax.ShapeDtypeStruct((N, DH), _f32),
        ],
    )(a2a, a2b, cnt0, cnt1, h1a, h1b, W2_l, b2, W2_r, fc_row)


def _tc3_body(pa_r, pb_r, fcb_r, out_r):
    out_r[...] = pa_r[...] + pb_r[...] + fcb_r[0, 0]


def _tc_final(pa, pb, fc_b11):
    return pl.pallas_call(
        _tc3_body,
        out_shape=jax.ShapeDtypeStruct((NCHUNKS, CH), _f32),
    )(pa, pb, fc_b11)


def kernel(x, edge_index, W1_l, b1, W1_r, W2_l, b2, W2_r, fc_W, fc_b):
    src = edge_index[0].astype(jnp.int32)
    dst = edge_index[1].astype(jnp.int32)

    z128 = jnp.zeros((RPS, D_IN), _f32)
    z8 = jnp.zeros((RPS, 8), _f32)
    ones8 = jnp.ones((CH, 8), _f32)

    agg0, agg1, cnt0, cnt1 = _sc_agg_l1(x, src, dst, z128, z8, ones8)
    h1a, h1b = _tc_layer1(agg0, agg1, cnt0, cnt1, x, W1_l,
                          b1.reshape(1, D_HID), W1_r)
    agg2a, agg2b = _sc_agg_l2(h1a, h1b, src, dst, z128)
    h2a, h2b, ga, gb = _tc_layer2(agg2a, agg2b, cnt0, cnt1, h1a, h1b,
                                  W2_l, b2.reshape(1, D_HID), W2_r,
                                  fc_W.reshape(1, D_HID))
    pa, pb = _sc_edge_dot(h2a, h2b, ga, gb, src, dst)
    out2 = _tc_final(pa, pb, fc_b.reshape(1, 1))
    return out2.reshape(E, 1)


# speedup vs baseline: 3.4842x; 3.4842x over previous
import functools

import jax
import jax.numpy as jnp
from jax import lax
from jax.experimental import pallas as pl
from jax.experimental.pallas import tpu as pltpu
from jax.experimental.pallas import tpu_sc as plsc

N = 10000
E = 320000
D_IN = 128
D_HID = 256
DH = 128

CH = 128
NCHUNKS = E // CH
NC, NS = 2, 16
NTILES = NC * NS
NPAD = 10240
RPS = NPAD // NS

_MESH = plsc.VectorSubcoreMesh(
    core_axis_name="c", subcore_axis_name="s", num_cores=NC, num_subcores=NS)

_f32 = jnp.float32

_PER32 = NCHUNKS // NTILES
_REM32 = NCHUNKS - _PER32 * NTILES
_PER16 = NCHUNKS // NS
_REM16 = NCHUNKS - _PER16 * NS


@functools.partial(
    pl.kernel,
    out_type=(
        jax.ShapeDtypeStruct((NPAD, 128), _f32),
        jax.ShapeDtypeStruct((NPAD, 128), _f32),
    ),
    mesh=_MESH,
    scratch_types=[
        pltpu.VMEM((CH,), jnp.int32),
        pltpu.VMEM((CH, 128), _f32),
        pltpu.VMEM_SHARED((NPAD, 128), _f32),
    ],
)
def _sc_cnt(dst_h, z128_h, ones_h, cnt0_h, cnt1_h, didx_v, ones_v, cnt_s):
    c = lax.axis_index("c")
    s = lax.axis_index("s")
    wid = c * NS + s
    base = pl.multiple_of(s * RPS, 8)
    pltpu.sync_copy(z128_h, cnt_s.at[pl.ds(base, RPS)])
    pltpu.sync_copy(ones_h, ones_v)
    plsc.subcore_barrier()

    nmine = jnp.where(wid < _REM32, _PER32 + 1, _PER32)

    def body(i, carry):
        j = wid + i * NTILES
        eb = pl.multiple_of(j * CH, 8)
        pltpu.sync_copy(dst_h.at[pl.ds(eb, CH)], didx_v)
        pltpu.sync_copy(ones_v, cnt_s.at[didx_v], add=True)
        return carry

    lax.fori_loop(0, nmine, body, 0)
    plsc.subcore_barrier()

    @pl.when(c == 0)
    def _():
        pltpu.sync_copy(cnt_s.at[pl.ds(base, RPS)], cnt0_h.at[pl.ds(base, RPS)])

    @pl.when(c == 1)
    def _():
        pltpu.sync_copy(cnt_s.at[pl.ds(base, RPS)], cnt1_h.at[pl.ds(base, RPS)])


@functools.partial(
    pl.kernel,
    out_type=(
        jax.ShapeDtypeStruct((NPAD, D_IN), _f32),
        jax.ShapeDtypeStruct((NPAD, D_IN), _f32),
    ),
    mesh=_MESH,
    scratch_types=[
        pltpu.VMEM((CH,), jnp.int32),
        pltpu.VMEM((CH,), jnp.int32),
        pltpu.VMEM((CH, D_IN), _f32),
        pltpu.VMEM_SHARED((NPAD, D_IN), _f32),
        pltpu.SemaphoreType.DMA,
    ],
)
def _sc_agg_l1(x_h, src_h, dst_h, z128_h,
               agg0_h, agg1_h,
               sidx_v, didx_v, rows_v, acc_s, sem):
    c = lax.axis_index("c")
    s = lax.axis_index("s")
    wid = c * NS + s
    base = pl.multiple_of(s * RPS, 8)
    pltpu.sync_copy(z128_h, acc_s.at[pl.ds(base, RPS)])
    plsc.subcore_barrier()

    nmine = jnp.where(wid < _REM32, _PER32 + 1, _PER32)

    def body(i, carry):
        j = wid + i * NTILES
        eb = pl.multiple_of(j * CH, 8)
        pltpu.sync_copy(src_h.at[pl.ds(eb, CH)], sidx_v)
        pltpu.sync_copy(dst_h.at[pl.ds(eb, CH)], didx_v)
        pltpu.async_copy(x_h.at[sidx_v], rows_v, sem).wait()
        pltpu.sync_copy(rows_v, acc_s.at[didx_v], add=True)
        return carry

    lax.fori_loop(0, nmine, body, 0)
    plsc.subcore_barrier()

    @pl.when(c == 0)
    def _():
        pltpu.sync_copy(acc_s.at[pl.ds(base, RPS)], agg0_h.at[pl.ds(base, RPS)])

    @pl.when(c == 1)
    def _():
        pltpu.sync_copy(acc_s.at[pl.ds(base, RPS)], agg1_h.at[pl.ds(base, RPS)])


@functools.partial(
    pl.kernel,
    out_type=(
        jax.ShapeDtypeStruct((NPAD, DH), _f32),
        jax.ShapeDtypeStruct((NPAD, DH), _f32),
    ),
    mesh=_MESH,
    scratch_types=[
        pltpu.VMEM((CH,), jnp.int32),
        pltpu.VMEM((CH,), jnp.int32),
        pltpu.VMEM((CH, DH), _f32),
        pltpu.VMEM_SHARED((NPAD, DH), _f32),
        pltpu.SemaphoreType.DMA,
    ],
)
def _sc_agg_l2(h1a_h, h1b_h, src_h, dst_h, z128_h,
               agg2a_h, agg2b_h,
               sidx_v, didx_v, rows_v, acc_s, sem):
    c = lax.axis_index("c")
    s = lax.axis_index("s")
    base = pl.multiple_of(s * RPS, 8)
    pltpu.sync_copy(z128_h, acc_s.at[pl.ds(base, RPS)])
    plsc.subcore_barrier()

    nmine = jnp.where(s < _REM16, _PER16 + 1, _PER16)

    def run(table_h, out_h):
        def body(i, carry):
            j = s + i * NS
            eb = pl.multiple_of(j * CH, 8)
            pltpu.sync_copy(src_h.at[pl.ds(eb, CH)], sidx_v)
            pltpu.sync_copy(dst_h.at[pl.ds(eb, CH)], didx_v)
            pltpu.async_copy(table_h.at[sidx_v], rows_v, sem).wait()
            pltpu.sync_copy(rows_v, acc_s.at[didx_v], add=True)
            return carry

        lax.fori_loop(0, nmine, body, 0)
        plsc.subcore_barrier()
        pltpu.sync_copy(acc_s.at[pl.ds(base, RPS)], out_h.at[pl.ds(base, RPS)])

    @pl.when(c == 0)
    def _():
        run(h1a_h, agg2a_h)

    @pl.when(c == 1)
    def _():
        run(h1b_h, agg2b_h)


@functools.partial(
    pl.kernel,
    out_type=(
        jax.ShapeDtypeStruct((E // 8, 128), _f32),
        jax.ShapeDtypeStruct((E // 8, 128), _f32),
    ),
    mesh=_MESH,
    scratch_types=[
        pltpu.VMEM((CH,), jnp.int32),
        pltpu.VMEM((CH,), jnp.int32),
        pltpu.VMEM((CH, DH), _f32),
        pltpu.VMEM((CH, DH), _f32),
        pltpu.VMEM((CH // 8, 128), _f32),
        pltpu.SemaphoreType.DMA,
        pltpu.SemaphoreType.DMA,
    ],
)
def _sc_edge_dot(h2a_h, h2b_h, ga_h, gb_h, src_h, dst_h,
                 pa_h, pb_h,
                 sidx_v, didx_v, hrows_v, grows_v, res_v, sem0, sem1):
    c = lax.axis_index("c")
    s = lax.axis_index("s")

    nmine = jnp.where(s < _REM16, _PER16 + 1, _PER16)

    def run(h_h, g_h, out_h):
        def body(i, carry):
            j = s + i * NS
            eb = pl.multiple_of(j * CH, 8)
            pltpu.sync_copy(src_h.at[pl.ds(eb, CH)], sidx_v)
            pltpu.sync_copy(dst_h.at[pl.ds(eb, CH)], didx_v)
            cp0 = pltpu.async_copy(h_h.at[sidx_v], hrows_v, sem0)
            cp1 = pltpu.async_copy(g_h.at[didx_v], grows_v, sem1)
            cp0.wait()
            cp1.wait()

            def rowloop(r, carry2):
                for k in range(8):
                    e = r * 8 + k
                    acc = hrows_v[e, pl.ds(0, 16)] * grows_v[e, pl.ds(0, 16)]
                    for q in range(1, DH // 16):
                        acc = acc + (hrows_v[e, pl.ds(q * 16, 16)] *
                                     grows_v[e, pl.ds(q * 16, 16)])
                    res_v[r, pl.ds(k * 16, 16)] = acc
                return carry2

            lax.fori_loop(0, CH // 8, rowloop, 0)
            rb = pl.multiple_of(j * (CH // 8), 8)
            pltpu.sync_copy(res_v, out_h.at[pl.ds(rb, CH // 8)])
            return carry

        lax.fori_loop(0, nmine, body, 0)

    @pl.when(c == 0)
    def _():
        run(h2a_h, ga_h, pa_h)

    @pl.when(c == 1)
    def _():
        run(h2b_h, gb_h, pb_h)


_NB = 400


def _tc1_body(agg0_r, agg1_r, cnt0_r, cnt1_r, x_r, wl_r, b_r, wr_r,
              h1a_r, h1b_r):
    cnt = cnt0_r[:, 0:1] + cnt1_r[:, 0:1]
    agg = agg0_r[...] + agg1_r[...]
    mean = agg / jnp.maximum(cnt, 1.0)
    h = (jnp.dot(mean, wl_r[...], preferred_element_type=_f32,
                 precision="highest")
         + jnp.dot(x_r[...], wr_r[...], preferred_element_type=_f32,
                   precision="highest")
         + b_r[...])
    h1a_r[...] = h[:, :DH]
    h1b_r[...] = h[:, DH:]


def _tc_layer1(agg0, agg1, cnt0, cnt1, x, W1_l, b1, W1_r):
    grid = N // _NB
    return pl.pallas_call(
        _tc1_body,
        grid=(grid,),
        in_specs=[
            pl.BlockSpec((_NB, D_IN), lambda i: (i, 0)),
            pl.BlockSpec((_NB, D_IN), lambda i: (i, 0)),
            pl.BlockSpec((_NB, 128), lambda i: (i, 0)),
            pl.BlockSpec((_NB, 128), lambda i: (i, 0)),
            pl.BlockSpec((_NB, D_IN), lambda i: (i, 0)),
            pl.BlockSpec((D_IN, D_HID), lambda i: (0, 0)),
            pl.BlockSpec((1, D_HID), lambda i: (0, 0)),
            pl.BlockSpec((D_IN, D_HID), lambda i: (0, 0)),
        ],
        out_specs=[
            pl.BlockSpec((_NB, DH), lambda i: (i, 0)),
            pl.BlockSpec((_NB, DH), lambda i: (i, 0)),
        ],
        out_shape=[
            jax.ShapeDtypeStruct((N, DH), _f32),
            jax.ShapeDtypeStruct((N, DH), _f32),
        ],
    )(agg0, agg1, cnt0, cnt1, x, W1_l, b1, W1_r)


def _tc2_body(a2a_r, a2b_r, cnt0_r, cnt1_r, h1a_r, h1b_r, wl_r, b_r, wr_r,
              fcr_r, h2a_r, h2b_r, ga_r, gb_r):
    cnt = jnp.maximum(cnt0_r[:, 0:1] + cnt1_r[:, 0:1], 1.0)
    mlo = a2a_r[...] / cnt
    mhi = a2b_r[...] / cnt
    kw = dict(preferred_element_type=_f32, precision="highest")
    h = (jnp.dot(mlo, wl_r[0:DH, :], **kw)
         + jnp.dot(mhi, wl_r[DH:, :], **kw)
         + jnp.dot(h1a_r[...], wr_r[0:DH, :], **kw)
         + jnp.dot(h1b_r[...], wr_r[DH:, :], **kw)
         + b_r[...])
    g = h * fcr_r[...]
    h2a_r[...] = h[:, :DH]
    h2b_r[...] = h[:, DH:]
    ga_r[...] = g[:, :DH]
    gb_r[...] = g[:, DH:]


def _tc_layer2(a2a, a2b, cnt0, cnt1, h1a, h1b, W2_l, b2, W2_r, fc_row):
    grid = N // _NB
    return pl.pallas_call(
        _tc2_body,
        grid=(grid,),
        in_specs=[
            pl.BlockSpec((_NB, DH), lambda i: (i, 0)),
            pl.BlockSpec((_NB, DH), lambda i: (i, 0)),
            pl.BlockSpec((_NB, 128), lambda i: (i, 0)),
            pl.BlockSpec((_NB, 128), lambda i: (i, 0)),
            pl.BlockSpec((_NB, DH), lambda i: (i, 0)),
            pl.BlockSpec((_NB, DH), lambda i: (i, 0)),
            pl.BlockSpec((D_HID, D_HID), lambda i: (0, 0)),
            pl.BlockSpec((1, D_HID), lambda i: (0, 0)),
            pl.BlockSpec((D_HID, D_HID), lambda i: (0, 0)),
            pl.BlockSpec((1, D_HID), lambda i: (0, 0)),
        ],
        out_specs=[
            pl.BlockSpec((_NB, DH), lambda i: (i, 0)),
            pl.BlockSpec((_NB, DH), lambda i: (i, 0)),
            pl.BlockSpec((_NB, DH), lambda i: (i, 0)),
            pl.BlockSpec((_NB, DH), lambda i: (i, 0)),
        ],
        out_shape=[
            jax.ShapeDtypeStruct((N, DH), _f32),
            jax.ShapeDtypeStruct((N, DH), _f32),
            jax.ShapeDtypeStruct((N, DH), _f32),
            jax.ShapeDtypeStruct((N, DH), _f32),
        ],
    )(a2a, a2b, cnt0, cnt1, h1a, h1b, W2_l, b2, W2_r, fc_row)


_EB = 4000


def _tc3_body(pa_r, pb_r, fcb_r, out_r):
    sm = pa_r[...] + pb_r[...]
    cols = [jnp.sum(sm[:, g * 16:(g + 1) * 16], axis=1, keepdims=True)
            for g in range(8)]
    out_r[...] = jnp.concatenate(cols, axis=1) + fcb_r[0, 0]


def _tc_final(pa, pb, fc_b11):
    return pl.pallas_call(
        _tc3_body,
        grid=(E // 8 // _EB,),
        in_specs=[
            pl.BlockSpec((_EB, 128), lambda i: (i, 0)),
            pl.BlockSpec((_EB, 128), lambda i: (i, 0)),
            pl.BlockSpec((1, 1), lambda i: (0, 0)),
        ],
        out_specs=pl.BlockSpec((_EB, 8), lambda i: (i, 0)),
        out_shape=jax.ShapeDtypeStruct((E // 8, 8), _f32),
    )(pa, pb, fc_b11)


def kernel(x, edge_index, W1_l, b1, W1_r, W2_l, b2, W2_r, fc_W, fc_b):
    src = edge_index[0].astype(jnp.int32)
    dst = edge_index[1].astype(jnp.int32)

    z128 = jnp.zeros((RPS, 128), _f32)
    ones128 = jnp.ones((CH, 128), _f32)

    cnt0, cnt1 = _sc_cnt(dst, z128, ones128)
    agg0, agg1 = _sc_agg_l1(x, src, dst, z128)
    h1a, h1b = _tc_layer1(agg0, agg1, cnt0, cnt1, x, W1_l,
                          b1.reshape(1, D_HID), W1_r)
    agg2a, agg2b = _sc_agg_l2(h1a, h1b, src, dst, z128)
    h2a, h2b, ga, gb = _tc_layer2(agg2a, agg2b, cnt0, cnt1, h1a, h1b,
                                  W2_l, b2.reshape(1, D_HID), W2_r,
                                  fc_W.reshape(1, D_HID))
    pa, pb = _sc_edge_dot(h2a, h2b, ga, gb, src, dst)
    out8 = _tc_final(pa, pb, fc_b.reshape(1, 1))
    return out8.reshape(E, 1)
